# Initial kernel scaffold; baseline (speedup 1.0000x reference)
import jax
import jax.numpy as jnp
from jax.experimental import pallas as pl


def kernel(x_src, x_target, edge_index, edge_type, target_node_type, weight, bias, relation_weight):
    raise NotImplementedError("write your pallas kernel here")


# speedup vs baseline: 9.4575x; 9.4575x over previous
import dataclasses
import functools

import jax
import jax.numpy as jnp
from jax import lax
from jax.experimental import pallas as pl
from jax.experimental.pallas import tpu as pltpu
from jax.experimental.pallas import tpu_sc as plsc

N_NODES = 10000
N_PAD = 10240
IN_CH = 128
OUT_CH = 128
NUM_T = 7
SCALING = 100.0
WIDTH = 144
NC = 2
NS = 16
NW = NC * NS
CHUNK = 128
ROW_BLK = 1000


def _scaled_table(x_src, weight, relation_weight):
    def body(rw_ref, x_ref, w_ref, out_ref, acc_ref):
        t = pl.program_id(1)

        @pl.when(t == 0)
        def _():
            acc_ref[...] = jnp.dot(
                x_ref[...], w_ref[...], preferred_element_type=jnp.float32
            )

        s = rw_ref[t] * SCALING
        s = jnp.where(s >= 0.0, s, 0.01 * s)
        out_ref[:, :OUT_CH] = acc_ref[...] * s
        out_ref[:, OUT_CH:] = jnp.full((ROW_BLK, WIDTH - OUT_CH), s, jnp.float32)

    n_blk = N_NODES // ROW_BLK
    return pl.pallas_call(
        body,
        grid=(n_blk, NUM_T),
        in_specs=[
            pl.BlockSpec(memory_space=pltpu.SMEM),
            pl.BlockSpec((ROW_BLK, IN_CH), lambda i, t: (i, 0)),
            pl.BlockSpec((IN_CH, OUT_CH), lambda i, t: (0, 0)),
        ],
        out_specs=pl.BlockSpec((ROW_BLK, WIDTH), lambda i, t: (t * n_blk + i, 0)),
        out_shape=jax.ShapeDtypeStruct((NUM_T * N_NODES, WIDTH), jnp.float32),
        scratch_shapes=[pltpu.VMEM((ROW_BLK, OUT_CH), jnp.float32)],
    )(relation_weight, x_src, weight)


def _sc_aggregate(table, row, col, ty, zeros):
    n_edges = row.shape[0]
    assert n_edges % CHUNK == 0
    n_chunks = n_edges // CHUNK
    n_iters = (n_chunks + NW - 1) // NW
    rows_per_sub = N_PAD // NS

    mesh = plsc.VectorSubcoreMesh(core_axis_name="c", subcore_axis_name="s")

    @functools.partial(
        pl.kernel,
        mesh=mesh,
        out_type=jax.ShapeDtypeStruct((NC, N_PAD, WIDTH), jnp.float32),
        scratch_types=[
            pltpu.VMEM((CHUNK,), jnp.int32),
            pltpu.VMEM((CHUNK,), jnp.int32),
            pltpu.VMEM((CHUNK,), jnp.int32),
            pltpu.VMEM((CHUNK,), jnp.int32),
            pltpu.VMEM((CHUNK, WIDTH), jnp.float32),
            pltpu.VMEM_SHARED((N_PAD, WIDTH), jnp.float32),
            pltpu.SemaphoreType.DMA,
        ],
        compiler_params=dataclasses.replace(
            pltpu.CompilerParams(), use_tc_tiling_on_sc=False
        ),
    )
    def k(table_hbm, row_hbm, col_hbm, ty_hbm, zeros_hbm, out_hbm,
          row_v, ty_v, col_v, gidx_v, rows_v, acc, sem):
        cid = lax.axis_index("c")
        sid = lax.axis_index("s")
        wid = sid * NC + cid

        sub_slc = pl.ds(sid * rows_per_sub, rows_per_sub)
        pltpu.sync_copy(zeros_hbm.at[sub_slc], acc.at[sub_slc])
        plsc.subcore_barrier()

        @pl.loop(0, n_iters)
        def _(j):
            ci = wid + j * NW

            @pl.when(ci < n_chunks)
            def _():
                eoff = ci * CHUNK
                eslc = pl.ds(eoff, CHUNK)
                pltpu.sync_copy(row_hbm.at[eslc], row_v)
                pltpu.sync_copy(ty_hbm.at[eslc], ty_v)
                pltpu.sync_copy(col_hbm.at[eslc], col_v)

                @pl.loop(0, CHUNK // 16)
                def _(kk):
                    sl = pl.ds(kk * 16, 16)
                    gidx_v[sl] = ty_v[sl] * N_NODES + row_v[sl]

                pltpu.async_copy(table_hbm.at[gidx_v], rows_v, sem).wait()
                pltpu.sync_copy(rows_v, acc.at[col_v], add=True)

        plsc.subcore_barrier()
        pltpu.sync_copy(acc.at[sub_slc], out_hbm.at[cid, sub_slc])

    return k(table, row, col, ty, zeros)


def _finalize(partial, bias):
    def body(p_ref, b_ref, o_ref):
        a = p_ref[0] + p_ref[1]
        deg = a[:, OUT_CH:OUT_CH + 1]
        inv = jnp.where(deg != 0.0, 1.0 / deg, 0.0)
        o_ref[...] = a[:, :OUT_CH] * inv + b_ref[...]

    n_blk = N_NODES // ROW_BLK
    return pl.pallas_call(
        body,
        grid=(n_blk,),
        in_specs=[
            pl.BlockSpec((NC, ROW_BLK, WIDTH), lambda i: (0, i, 0)),
            pl.BlockSpec((1, OUT_CH), lambda i: (0, 0)),
        ],
        out_specs=pl.BlockSpec((ROW_BLK, OUT_CH), lambda i: (i, 0)),
        out_shape=jax.ShapeDtypeStruct((N_NODES, OUT_CH), jnp.float32),
    )(partial, bias)


def kernel(x_src, x_target, edge_index, edge_type, target_node_type,
           weight, bias, relation_weight):
    row = edge_index[0].astype(jnp.int32)
    col = edge_index[1].astype(jnp.int32)
    ty = edge_type.astype(jnp.int32)
    table = _scaled_table(x_src, weight, relation_weight.astype(jnp.float32))
    zeros = jnp.zeros((N_PAD, WIDTH), jnp.float32)
    partial = _sc_aggregate(table, row, col, ty, zeros)
    return _finalize(partial, bias.reshape(1, OUT_CH))
